# baseline (speedup 1.0000x reference)
import functools

import jax
import jax.numpy as jnp
from jax import lax
from jax.experimental import pallas as pl
from jax.experimental.pallas import tpu as pltpu
from jax.experimental.pallas import tpu_sc as plsc

N_NODES = 10000
N_EDGES = 320000
D_FEAT = 128
D_EDGE = 16

NC = 2
NS = 16
NW = NC * NS
EW = N_EDGES // NW
BL = 128
NBL = EW // BL
TAIL = EW - NBL * BL
NPAIR = NBL // 2
NP = 10240
SROW = NP // NS
IDXR = (EW + BL - 1) // BL + 1


def _mesh():
    return plsc.VectorSubcoreMesh(core_axis_name="c", subcore_axis_name="s",
                                  num_cores=NC, num_subcores=NS)


def _wid():
    return lax.axis_index("s") * NC + lax.axis_index("c")


def _sc_gather(x, src_r):
    @functools.partial(
        pl.kernel,
        out_type=jax.ShapeDtypeStruct((N_EDGES, D_FEAT), jnp.float32),
        mesh=_mesh(),
        scratch_types=[
            pltpu.VMEM((IDXR, BL), jnp.int32),
            pltpu.VMEM((BL, D_FEAT), jnp.float32),
            pltpu.VMEM((BL, D_FEAT), jnp.float32),
            pltpu.SemaphoreType.DMA,
            pltpu.SemaphoreType.DMA,
            pltpu.SemaphoreType.DMA,
            pltpu.SemaphoreType.DMA,
        ],
    )
    def k(x_hbm, src_hbm, xs_hbm, idx_v, rows0, rows1, g0, g1, w0, w1):
        wid = _wid()
        base = wid * EW
        pltpu.sync_copy(src_hbm.at[wid], idx_v)

        def gat(j, buf, sem):
            pltpu.async_copy(x_hbm.at[idx_v.at[j]], buf, sem)

        def wait_g(buf, sem):
            pltpu.make_async_copy(xs_hbm.at[pl.ds(0, BL)], buf, sem).wait()

        def wr(j, buf, sem):
            pltpu.async_copy(buf, xs_hbm.at[pl.ds(base + j * BL, BL)], sem)

        def wait_w(buf, sem):
            pltpu.make_async_copy(buf, xs_hbm.at[pl.ds(0, BL)], sem).wait()

        gat(0, rows0, g0)

        def body(jj, carry):
            j0 = 2 * jj
            gat(j0 + 1, rows1, g1)
            wait_g(rows0, g0)
            wr(j0, rows0, w0)

            @pl.when(jj < NPAIR - 1)
            def _():
                wait_w(rows0, w0)
                gat(j0 + 2, rows0, g0)

            wait_g(rows1, g1)
            wr(j0 + 1, rows1, w1)
            wait_w(rows1, w1)
            return carry

        lax.fori_loop(0, NPAIR, body, 0)
        wait_w(rows0, w0)
        tg = pltpu.async_copy(
            x_hbm.at[idx_v.at[NBL, pl.ds(0, TAIL)]],
            rows0.at[pl.ds(0, TAIL)], g0)
        tg.wait()
        pltpu.sync_copy(rows0.at[pl.ds(0, TAIL)],
                        xs_hbm.at[pl.ds(base + NBL * BL, TAIL)])

    return k(x, src_r)


def _tc_edge(xs, edge_attr, W1, b1, W2, b2, att_vec):
    PK = 8
    PR = 800
    EP = N_EDGES // PK
    grid = (EP // PR,)

    xs2 = xs.reshape(EP, PK * D_FEAT)
    ea2 = edge_attr.reshape(EP, PK * D_EDGE)
    eye = jnp.eye(PK, dtype=jnp.float32)
    S = jnp.kron(eye, jnp.ones((D_EDGE, D_EDGE), jnp.float32))
    W1rep = jnp.kron(eye, W1)
    attrep = jnp.kron(eye, att_vec)
    Bm = jnp.kron(eye, jnp.ones((1, D_FEAT), jnp.float32))
    b1rep = jnp.tile(b1, PK)
    b2rep = jnp.tile(b2, PK)

    def body(xs_ref, ea_ref, s_ref, w1_ref, b1_ref, w2_ref, b2_ref,
             att_ref, bm_ref, wmsg_ref, e_ref):
        ea = ea_ref[...]
        sq = ea * ea
        n2b = jnp.dot(sq, s_ref[...], preferred_element_type=jnp.float32)
        rinv = 1.0 / (jnp.sqrt(n2b) + 1e-8)
        ean = ea * rinv
        h = jnp.tanh(
            jnp.dot(ean.astype(jnp.bfloat16),
                    w1_ref[...].astype(jnp.bfloat16),
                    preferred_element_type=jnp.float32)
            + b1_ref[...])
        w2 = w2_ref[...].astype(jnp.bfloat16)
        hb = h.astype(jnp.bfloat16)
        filt = jnp.concatenate(
            [jnp.dot(hb[:, k * D_FEAT:(k + 1) * D_FEAT], w2,
                     preferred_element_type=jnp.float32)
             for k in range(PK)], axis=1) + b2_ref[...]
        msg = xs_ref[...] * filt
        s8 = jnp.dot(msg, att_ref[...], preferred_element_type=jnp.float32)
        e8 = jnp.exp(s8)
        ebc = jnp.dot(e8, bm_ref[...], preferred_element_type=jnp.float32)
        wmsg_ref[...] = msg * ebc
        e_ref[...] = e8

    wmsg2, e2 = pl.pallas_call(
        body,
        grid=grid,
        in_specs=[
            pl.BlockSpec((PR, PK * D_FEAT), lambda i: (i, 0)),
            pl.BlockSpec((PR, PK * D_EDGE), lambda i: (i, 0)),
            pl.BlockSpec((PK * D_EDGE, PK * D_EDGE), lambda i: (0, 0)),
            pl.BlockSpec((PK * D_EDGE, PK * D_FEAT), lambda i: (0, 0)),
            pl.BlockSpec((PK * D_FEAT,), lambda i: (0,)),
            pl.BlockSpec((D_FEAT, D_FEAT), lambda i: (0, 0)),
            pl.BlockSpec((PK * D_FEAT,), lambda i: (0,)),
            pl.BlockSpec((PK * D_FEAT, PK), lambda i: (0, 0)),
            pl.BlockSpec((PK, PK * D_FEAT), lambda i: (0, 0)),
        ],
        out_specs=[
            pl.BlockSpec((PR, PK * D_FEAT), lambda i: (i, 0)),
            pl.BlockSpec((PR, PK), lambda i: (i, 0)),
        ],
        out_shape=[
            jax.ShapeDtypeStruct((EP, PK * D_FEAT), jnp.float32),
            jax.ShapeDtypeStruct((EP, PK), jnp.float32),
        ],
    )(xs2, ea2, S, W1rep, b1rep, W2, b2rep, attrep, Bm)
    return wmsg2.reshape(N_EDGES, D_FEAT), e2.reshape(N_EDGES)


def _sc_scatter(wmsg, e, tgt_r):
    @functools.partial(
        pl.kernel,
        out_type=[
            jax.ShapeDtypeStruct((NC, NP, D_FEAT), jnp.float32),
            jax.ShapeDtypeStruct((NC, NP), jnp.float32),
        ],
        mesh=_mesh(),
        scratch_types=[
            pltpu.VMEM((IDXR, BL), jnp.int32),
            pltpu.VMEM((BL, D_FEAT), jnp.float32),
            pltpu.VMEM((BL, D_FEAT), jnp.float32),
            pltpu.VMEM((BL,), jnp.float32),
            pltpu.VMEM((BL,), jnp.float32),
            pltpu.VMEM((32, D_FEAT), jnp.float32),
            pltpu.VMEM((SROW,), jnp.float32),
            pltpu.VMEM_SHARED((NP, D_FEAT), jnp.float32),
            pltpu.VMEM_SHARED((NP,), jnp.float32),
            pltpu.SemaphoreType.DMA,
            pltpu.SemaphoreType.DMA,
        ],
    )
    def k(wmsg_hbm, e_hbm, tgt_hbm, a2_hbm, z2_hbm,
          idx_v, m0, m1, e0, e1, zb_v, zb1_v, a_sh, z_sh, r0, r1):
        c = lax.axis_index("c")
        s = lax.axis_index("s")
        wid = s * NC + c
        base = wid * EW

        zeros16 = jnp.zeros((16,), jnp.float32)

        def z0(i, carry):
            def z1(kk, carry2):
                zb_v[i, pl.ds(kk * 16, 16)] = zeros16
                return carry2
            return lax.fori_loop(0, 8, z1, carry)

        lax.fori_loop(0, 32, z0, 0)

        def z2(i, carry):
            zb1_v[pl.ds(i * 16, 16)] = zeros16
            return carry

        lax.fori_loop(0, SROW // 16, z2, 0)

        def za(t, carry):
            pltpu.sync_copy(zb_v, a_sh.at[pl.ds(s * SROW + t * 32, 32)])
            return carry

        lax.fori_loop(0, SROW // 32, za, 0)
        pltpu.sync_copy(zb1_v, z_sh.at[pl.ds(s * SROW, SROW)])

        pltpu.sync_copy(tgt_hbm.at[wid], idx_v)
        plsc.subcore_barrier()

        def rd(j, mb, eb, sem):
            pltpu.async_copy(wmsg_hbm.at[pl.ds(base + j * BL, BL)], mb, sem)
            pltpu.async_copy(e_hbm.at[pl.ds(base + j * BL, BL)], eb, sem)

        def wait_rd(mb, eb, sem):
            pltpu.make_async_copy(wmsg_hbm.at[pl.ds(0, BL)], mb, sem).wait()
            pltpu.make_async_copy(e_hbm.at[pl.ds(0, BL)], eb, sem).wait()

        def scat(j, mb, eb):
            pltpu.sync_copy(mb, a_sh.at[idx_v.at[j]], add=True)
            pltpu.sync_copy(eb, z_sh.at[idx_v.at[j]], add=True)

        rd(0, m0, e0, r0)

        def body(jj, carry):
            j0 = 2 * jj
            rd(j0 + 1, m1, e1, r1)
            wait_rd(m0, e0, r0)
            scat(j0, m0, e0)

            @pl.when(jj < NPAIR - 1)
            def _():
                rd(j0 + 2, m0, e0, r0)

            wait_rd(m1, e1, r1)
            scat(j0 + 1, m1, e1)
            return carry

        lax.fori_loop(0, NPAIR, body, 0)
        pltpu.sync_copy(wmsg_hbm.at[pl.ds(base + NBL * BL, TAIL)],
                        m0.at[pl.ds(0, TAIL)])
        pltpu.sync_copy(e_hbm.at[pl.ds(base + NBL * BL, TAIL)],
                        e0.at[pl.ds(0, TAIL)])
        pltpu.sync_copy(m0.at[pl.ds(0, TAIL)],
                        a_sh.at[idx_v.at[NBL, pl.ds(0, TAIL)]], add=True)
        pltpu.sync_copy(e0.at[pl.ds(0, TAIL)],
                        z_sh.at[idx_v.at[NBL, pl.ds(0, TAIL)]], add=True)

        plsc.subcore_barrier()
        pltpu.sync_copy(a_sh.at[pl.ds(s * SROW, SROW)],
                        a2_hbm.at[c, pl.ds(s * SROW, SROW)])
        pltpu.sync_copy(z_sh.at[pl.ds(s * SROW, SROW)],
                        z2_hbm.at[c, pl.ds(s * SROW, SROW)])

    return k(wmsg, e, tgt_r)


def _tc_finalize(a2, z2):
    BN = 1024
    grid = (NP // BN,)

    def body(a_ref, z_ref, out_ref, zc_ref):
        a = a_ref[0] + a_ref[1]
        z = z_ref[0] + z_ref[1]
        zc_ref[...] = z
        out_ref[...] = a / (z[:, None] + 1e-16)

    outp, zc = pl.pallas_call(
        body,
        grid=grid,
        in_specs=[
            pl.BlockSpec((NC, BN, D_FEAT), lambda i: (0, i, 0)),
            pl.BlockSpec((NC, BN), lambda i: (0, i)),
        ],
        out_specs=[
            pl.BlockSpec((BN, D_FEAT), lambda i: (i, 0)),
            pl.BlockSpec((BN,), lambda i: (i,)),
        ],
        out_shape=[
            jax.ShapeDtypeStruct((NP, D_FEAT), jnp.float32),
            jax.ShapeDtypeStruct((NP,), jnp.float32),
        ],
    )(a2, z2)
    return outp, zc


def _sc_weights(e, zc, tgt_r):
    @functools.partial(
        pl.kernel,
        out_type=jax.ShapeDtypeStruct((N_EDGES,), jnp.float32),
        mesh=_mesh(),
        scratch_types=[
            pltpu.VMEM((IDXR, BL), jnp.int32),
            pltpu.VMEM((BL,), jnp.float32),
            pltpu.VMEM((BL,), jnp.float32),
            pltpu.VMEM((BL,), jnp.float32),
            pltpu.VMEM((BL,), jnp.float32),
            pltpu.VMEM((BL,), jnp.float32),
            pltpu.VMEM((BL,), jnp.float32),
            pltpu.SemaphoreType.DMA,
            pltpu.SemaphoreType.DMA,
            pltpu.SemaphoreType.DMA,
            pltpu.SemaphoreType.DMA,
        ],
    )
    def k(e_hbm, zc_hbm, tgt_hbm, w_hbm,
          idx_v, ev0, ev1, zv0, zv1, wv0, wv1, r0, r1, w0, w1):
        wid = _wid()
        base = wid * EW
        pltpu.sync_copy(tgt_hbm.at[wid], idx_v)

        def rd(j, eb, zb, sem):
            pltpu.async_copy(e_hbm.at[pl.ds(base + j * BL, BL)], eb, sem)
            pltpu.async_copy(zc_hbm.at[idx_v.at[j]], zb, sem)

        def wait_rd(eb, zb, sem):
            pltpu.make_async_copy(e_hbm.at[pl.ds(0, BL)], eb, sem).wait()
            pltpu.make_async_copy(e_hbm.at[pl.ds(0, BL)], zb, sem).wait()

        def comp(eb, zb, wb):
            for kk in range(BL // 16):
                sl = pl.ds(kk * 16, 16)
                wb[sl] = eb[sl] / (zb[sl] + 1e-16)

        def wr(j, wb, sem):
            pltpu.async_copy(wb, w_hbm.at[pl.ds(base + j * BL, BL)], sem)

        def wait_w(wb, sem):
            pltpu.make_async_copy(wb, w_hbm.at[pl.ds(0, BL)], sem).wait()

        rd(0, ev0, zv0, r0)

        def body(jj, carry):
            j0 = 2 * jj
            rd(j0 + 1, ev1, zv1, r1)
            wait_rd(ev0, zv0, r0)

            @pl.when(jj > 0)
            def _():
                wait_w(wv0, w0)

            comp(ev0, zv0, wv0)
            wr(j0, wv0, w0)

            @pl.when(jj < NPAIR - 1)
            def _():
                rd(j0 + 2, ev0, zv0, r0)

            wait_rd(ev1, zv1, r1)

            @pl.when(jj > 0)
            def _():
                wait_w(wv1, w1)

            comp(ev1, zv1, wv1)
            wr(j0 + 1, wv1, w1)
            return carry

        lax.fori_loop(0, NPAIR, body, 0)
        wait_w(wv0, w0)
        wait_w(wv1, w1)
        pltpu.sync_copy(e_hbm.at[pl.ds(base + NBL * BL, TAIL)],
                        ev0.at[pl.ds(0, TAIL)])
        pltpu.async_copy(zc_hbm.at[idx_v.at[NBL, pl.ds(0, TAIL)]],
                         zv0.at[pl.ds(0, TAIL)], r0).wait()
        wv0[pl.ds(0, 16)] = ev0[pl.ds(0, 16)] / (zv0[pl.ds(0, 16)] + 1e-16)
        pltpu.sync_copy(wv0.at[pl.ds(0, TAIL)],
                        w_hbm.at[pl.ds(base + NBL * BL, TAIL)])

    return k(e, zc, tgt_r)


def kernel(x, edge_index, edge_attr, W1, b1, W2, b2, att_vec):
    src = edge_index[0].astype(jnp.int32)
    tgt = edge_index[1].astype(jnp.int32)
    src_r = jnp.pad(src.reshape(NW, EW), ((0, 0), (0, IDXR * BL - EW))
                    ).reshape(NW, IDXR, BL)
    tgt_r = jnp.pad(tgt.reshape(NW, EW), ((0, 0), (0, IDXR * BL - EW))
                    ).reshape(NW, IDXR, BL)

    xs = _sc_gather(x, src_r)
    wmsg, e = _tc_edge(xs, edge_attr, W1, b1, W2, b2, att_vec)
    a2, z2 = _sc_scatter(wmsg, e, tgt_r)
    outp, zc = _tc_finalize(a2, z2)
    w = _sc_weights(e, zc, tgt_r)
    return outp[:N_NODES], w


# speedup vs baseline: 1.0040x; 1.0040x over previous
import functools

import jax
import jax.numpy as jnp
from jax import lax
from jax.experimental import pallas as pl
from jax.experimental.pallas import tpu as pltpu
from jax.experimental.pallas import tpu_sc as plsc

N_NODES = 10000
N_EDGES = 320000
D_FEAT = 128
D_EDGE = 16

NC = 2
NS = 16
NW = NC * NS
EW = N_EDGES // NW
BL = 128
NBL = EW // BL
TAIL = EW - NBL * BL
NPAIR = NBL // 2
NP = 10240
SROW = NP // NS
IDXR = (EW + BL - 1) // BL + 1


def _mesh():
    return plsc.VectorSubcoreMesh(core_axis_name="c", subcore_axis_name="s",
                                  num_cores=NC, num_subcores=NS)


def _wid():
    return lax.axis_index("s") * NC + lax.axis_index("c")


def _sc_gather(x, src_r):
    @functools.partial(
        pl.kernel,
        out_type=jax.ShapeDtypeStruct((N_EDGES, D_FEAT), jnp.float32),
        mesh=_mesh(),
        scratch_types=[
            pltpu.VMEM((IDXR, BL), jnp.int32),
            pltpu.VMEM((BL, D_FEAT), jnp.float32),
            pltpu.VMEM((BL, D_FEAT), jnp.float32),
            pltpu.SemaphoreType.DMA,
            pltpu.SemaphoreType.DMA,
            pltpu.SemaphoreType.DMA,
            pltpu.SemaphoreType.DMA,
        ],
    )
    def k(x_hbm, src_hbm, xs_hbm, idx_v, rows0, rows1, g0, g1, w0, w1):
        wid = _wid()
        base = wid * EW
        pltpu.sync_copy(src_hbm.at[wid], idx_v)

        def gat(j, buf, sem):
            pltpu.async_copy(x_hbm.at[idx_v.at[j]], buf, sem)

        def wait_g(buf, sem):
            pltpu.make_async_copy(xs_hbm.at[pl.ds(0, BL)], buf, sem).wait()

        def wr(j, buf, sem):
            pltpu.async_copy(buf, xs_hbm.at[pl.ds(base + j * BL, BL)], sem)

        def wait_w(buf, sem):
            pltpu.make_async_copy(buf, xs_hbm.at[pl.ds(0, BL)], sem).wait()

        gat(0, rows0, g0)

        def body(jj, carry):
            j0 = 2 * jj
            gat(j0 + 1, rows1, g1)
            wait_g(rows0, g0)
            wr(j0, rows0, w0)

            @pl.when(jj < NPAIR - 1)
            def _():
                wait_w(rows0, w0)
                gat(j0 + 2, rows0, g0)

            wait_g(rows1, g1)
            wr(j0 + 1, rows1, w1)
            wait_w(rows1, w1)
            return carry

        lax.fori_loop(0, NPAIR, body, 0)
        wait_w(rows0, w0)
        tg = pltpu.async_copy(
            x_hbm.at[idx_v.at[NBL, pl.ds(0, TAIL)]],
            rows0.at[pl.ds(0, TAIL)], g0)
        tg.wait()
        pltpu.sync_copy(rows0.at[pl.ds(0, TAIL)],
                        xs_hbm.at[pl.ds(base + NBL * BL, TAIL)])

    return k(x, src_r)


def _tc_edge(xs, edge_attr, W1, b1, W2, b2, att_vec):
    PK = 8
    PR = 800
    EP = N_EDGES // PK
    DP = PK * D_FEAT
    grid = (EP // PR,)
    ER = PR * PK // D_FEAT

    xs2 = xs.reshape(EP, DP)
    ea2 = edge_attr.reshape(EP, PK * D_EDGE)

    def body(xs_ref, ea_ref, w1_ref, b1_ref, w2_ref, b2_ref, att_ref,
             wmsg_ref, e_ref, s_s, w1_s, att_s, bm_s):
        @pl.when(pl.program_id(0) == 0)
        def _():
            gi = lax.broadcasted_iota(jnp.int32, (128, 128), 0) // D_EDGE
            gj = lax.broadcasted_iota(jnp.int32, (128, 128), 1) // D_EDGE
            s_s[...] = jnp.where(gi == gj, 1.0, 0.0)
            w1t = jnp.concatenate([w1_ref[...]] * PK, axis=0)
            w1big = jnp.concatenate([w1t] * PK, axis=1)
            ri = lax.broadcasted_iota(jnp.int32, (128, DP), 0) // D_EDGE
            cj = lax.broadcasted_iota(jnp.int32, (128, DP), 1) // D_FEAT
            w1_s[...] = jnp.where(
                ri == cj, w1big, 0.0).astype(jnp.bfloat16)
            attt = jnp.concatenate([att_ref[...]] * PK, axis=0)
            ai = lax.broadcasted_iota(jnp.int32, (DP, PK), 0) // D_FEAT
            aj = lax.broadcasted_iota(jnp.int32, (DP, PK), 1)
            att_s[...] = jnp.where(ai == aj, attt, 0.0)
            bi = lax.broadcasted_iota(jnp.int32, (PK, DP), 0)
            bj = lax.broadcasted_iota(jnp.int32, (PK, DP), 1) // D_FEAT
            bm_s[...] = jnp.where(bi == bj, 1.0, 0.0)

        b1rep = jnp.concatenate([b1_ref[...]] * PK)
        b2rep = jnp.concatenate([b2_ref[...]] * PK)
        ea = ea_ref[...]
        sq = ea * ea
        n2b = jnp.dot(sq, s_s[...], preferred_element_type=jnp.float32)
        rinv = 1.0 / (jnp.sqrt(n2b) + 1e-8)
        ean = ea * rinv
        h = jnp.tanh(
            jnp.dot(ean.astype(jnp.bfloat16), w1_s[...],
                    preferred_element_type=jnp.float32)
            + b1rep)
        w2 = w2_ref[...].astype(jnp.bfloat16)
        hb = h.astype(jnp.bfloat16)
        filt = jnp.concatenate(
            [jnp.dot(hb[:, k * D_FEAT:(k + 1) * D_FEAT], w2,
                     preferred_element_type=jnp.float32)
             for k in range(PK)], axis=1) + b2rep
        msg = xs_ref[...] * filt
        s8 = jnp.dot(msg, att_s[...], preferred_element_type=jnp.float32)
        e8 = jnp.exp(s8)
        ebc = jnp.dot(e8, bm_s[...], preferred_element_type=jnp.float32)
        wmsg_ref[...] = msg * ebc
        e_ref[...] = e8

    wmsg2, e2 = pl.pallas_call(
        body,
        grid=grid,
        in_specs=[
            pl.BlockSpec((PR, DP), lambda i: (i, 0)),
            pl.BlockSpec((PR, PK * D_EDGE), lambda i: (i, 0)),
            pl.BlockSpec((D_EDGE, D_FEAT), lambda i: (0, 0)),
            pl.BlockSpec((D_FEAT,), lambda i: (0,)),
            pl.BlockSpec((D_FEAT, D_FEAT), lambda i: (0, 0)),
            pl.BlockSpec((D_FEAT,), lambda i: (0,)),
            pl.BlockSpec((D_FEAT, 1), lambda i: (0, 0)),
        ],
        out_specs=[
            pl.BlockSpec((PR, DP), lambda i: (i, 0)),
            pl.BlockSpec((PR, PK), lambda i: (i, 0)),
        ],
        out_shape=[
            jax.ShapeDtypeStruct((EP, DP), jnp.float32),
            jax.ShapeDtypeStruct((EP, PK), jnp.float32),
        ],
        scratch_shapes=[
            pltpu.VMEM((128, 128), jnp.float32),
            pltpu.VMEM((128, DP), jnp.bfloat16),
            pltpu.VMEM((DP, PK), jnp.float32),
            pltpu.VMEM((PK, DP), jnp.float32),
        ],
    )(xs2, ea2, W1, b1, W2, b2, att_vec)
    return wmsg2.reshape(N_EDGES, D_FEAT), e2.reshape(N_EDGES)


def _sc_scatter(wmsg, e, tgt_r):
    @functools.partial(
        pl.kernel,
        out_type=[
            jax.ShapeDtypeStruct((NC, NP, D_FEAT), jnp.float32),
            jax.ShapeDtypeStruct((NC, NP), jnp.float32),
        ],
        mesh=_mesh(),
        scratch_types=[
            pltpu.VMEM((IDXR, BL), jnp.int32),
            pltpu.VMEM((BL, D_FEAT), jnp.float32),
            pltpu.VMEM((BL, D_FEAT), jnp.float32),
            pltpu.VMEM((BL,), jnp.float32),
            pltpu.VMEM((BL,), jnp.float32),
            pltpu.VMEM((32, D_FEAT), jnp.float32),
            pltpu.VMEM((SROW,), jnp.float32),
            pltpu.VMEM_SHARED((NP, D_FEAT), jnp.float32),
            pltpu.VMEM_SHARED((NP,), jnp.float32),
            pltpu.SemaphoreType.DMA,
            pltpu.SemaphoreType.DMA,
        ],
    )
    def k(wmsg_hbm, e_hbm, tgt_hbm, a2_hbm, z2_hbm,
          idx_v, m0, m1, e0, e1, zb_v, zb1_v, a_sh, z_sh, r0, r1):
        c = lax.axis_index("c")
        s = lax.axis_index("s")
        wid = s * NC + c
        base = wid * EW

        zeros16 = jnp.zeros((16,), jnp.float32)

        def z0(i, carry):
            def z1(kk, carry2):
                zb_v[i, pl.ds(kk * 16, 16)] = zeros16
                return carry2
            return lax.fori_loop(0, 8, z1, carry)

        lax.fori_loop(0, 32, z0, 0)

        def z2(i, carry):
            zb1_v[pl.ds(i * 16, 16)] = zeros16
            return carry

        lax.fori_loop(0, SROW // 16, z2, 0)

        def za(t, carry):
            pltpu.sync_copy(zb_v, a_sh.at[pl.ds(s * SROW + t * 32, 32)])
            return carry

        lax.fori_loop(0, SROW // 32, za, 0)
        pltpu.sync_copy(zb1_v, z_sh.at[pl.ds(s * SROW, SROW)])

        pltpu.sync_copy(tgt_hbm.at[wid], idx_v)
        plsc.subcore_barrier()

        def rd(j, mb, eb, sem):
            pltpu.async_copy(wmsg_hbm.at[pl.ds(base + j * BL, BL)], mb, sem)
            pltpu.async_copy(e_hbm.at[pl.ds(base + j * BL, BL)], eb, sem)

        def wait_rd(mb, eb, sem):
            pltpu.make_async_copy(wmsg_hbm.at[pl.ds(0, BL)], mb, sem).wait()
            pltpu.make_async_copy(e_hbm.at[pl.ds(0, BL)], eb, sem).wait()

        def scat(j, mb, eb):
            pltpu.sync_copy(mb, a_sh.at[idx_v.at[j]], add=True)
            pltpu.sync_copy(eb, z_sh.at[idx_v.at[j]], add=True)

        rd(0, m0, e0, r0)

        def body(jj, carry):
            j0 = 2 * jj
            rd(j0 + 1, m1, e1, r1)
            wait_rd(m0, e0, r0)
            scat(j0, m0, e0)

            @pl.when(jj < NPAIR - 1)
            def _():
                rd(j0 + 2, m0, e0, r0)

            wait_rd(m1, e1, r1)
            scat(j0 + 1, m1, e1)
            return carry

        lax.fori_loop(0, NPAIR, body, 0)
        pltpu.sync_copy(wmsg_hbm.at[pl.ds(base + NBL * BL, TAIL)],
                        m0.at[pl.ds(0, TAIL)])
        pltpu.sync_copy(e_hbm.at[pl.ds(base + NBL * BL, TAIL)],
                        e0.at[pl.ds(0, TAIL)])
        pltpu.sync_copy(m0.at[pl.ds(0, TAIL)],
                        a_sh.at[idx_v.at[NBL, pl.ds(0, TAIL)]], add=True)
        pltpu.sync_copy(e0.at[pl.ds(0, TAIL)],
                        z_sh.at[idx_v.at[NBL, pl.ds(0, TAIL)]], add=True)

        plsc.subcore_barrier()
        pltpu.sync_copy(a_sh.at[pl.ds(s * SROW, SROW)],
                        a2_hbm.at[c, pl.ds(s * SROW, SROW)])
        pltpu.sync_copy(z_sh.at[pl.ds(s * SROW, SROW)],
                        z2_hbm.at[c, pl.ds(s * SROW, SROW)])

    return k(wmsg, e, tgt_r)


def _tc_finalize(a2, z2):
    BN = 1024
    grid = (NP // BN,)

    def body(a_ref, z_ref, out_ref, zc_ref):
        a = a_ref[0] + a_ref[1]
        z = z_ref[0] + z_ref[1]
        zc_ref[...] = z
        out_ref[...] = a / (z[:, None] + 1e-16)

    outp, zc = pl.pallas_call(
        body,
        grid=grid,
        in_specs=[
            pl.BlockSpec((NC, BN, D_FEAT), lambda i: (0, i, 0)),
            pl.BlockSpec((NC, BN), lambda i: (0, i)),
        ],
        out_specs=[
            pl.BlockSpec((BN, D_FEAT), lambda i: (i, 0)),
            pl.BlockSpec((BN,), lambda i: (i,)),
        ],
        out_shape=[
            jax.ShapeDtypeStruct((NP, D_FEAT), jnp.float32),
            jax.ShapeDtypeStruct((NP,), jnp.float32),
        ],
    )(a2, z2)
    return outp, zc


def _sc_weights(e, zc, tgt_r):
    @functools.partial(
        pl.kernel,
        out_type=jax.ShapeDtypeStruct((N_EDGES,), jnp.float32),
        mesh=_mesh(),
        scratch_types=[
            pltpu.VMEM((IDXR, BL), jnp.int32),
            pltpu.VMEM((BL,), jnp.float32),
            pltpu.VMEM((BL,), jnp.float32),
            pltpu.VMEM((BL,), jnp.float32),
            pltpu.VMEM((BL,), jnp.float32),
            pltpu.VMEM((BL,), jnp.float32),
            pltpu.VMEM((BL,), jnp.float32),
            pltpu.SemaphoreType.DMA,
            pltpu.SemaphoreType.DMA,
            pltpu.SemaphoreType.DMA,
            pltpu.SemaphoreType.DMA,
        ],
    )
    def k(e_hbm, zc_hbm, tgt_hbm, w_hbm,
          idx_v, ev0, ev1, zv0, zv1, wv0, wv1, r0, r1, w0, w1):
        wid = _wid()
        base = wid * EW
        pltpu.sync_copy(tgt_hbm.at[wid], idx_v)

        def rd(j, eb, zb, sem):
            pltpu.async_copy(e_hbm.at[pl.ds(base + j * BL, BL)], eb, sem)
            pltpu.async_copy(zc_hbm.at[idx_v.at[j]], zb, sem)

        def wait_rd(eb, zb, sem):
            pltpu.make_async_copy(e_hbm.at[pl.ds(0, BL)], eb, sem).wait()
            pltpu.make_async_copy(e_hbm.at[pl.ds(0, BL)], zb, sem).wait()

        def comp(eb, zb, wb):
            for kk in range(BL // 16):
                sl = pl.ds(kk * 16, 16)
                wb[sl] = eb[sl] / (zb[sl] + 1e-16)

        def wr(j, wb, sem):
            pltpu.async_copy(wb, w_hbm.at[pl.ds(base + j * BL, BL)], sem)

        def wait_w(wb, sem):
            pltpu.make_async_copy(wb, w_hbm.at[pl.ds(0, BL)], sem).wait()

        rd(0, ev0, zv0, r0)

        def body(jj, carry):
            j0 = 2 * jj
            rd(j0 + 1, ev1, zv1, r1)
            wait_rd(ev0, zv0, r0)

            @pl.when(jj > 0)
            def _():
                wait_w(wv0, w0)

            comp(ev0, zv0, wv0)
            wr(j0, wv0, w0)

            @pl.when(jj < NPAIR - 1)
            def _():
                rd(j0 + 2, ev0, zv0, r0)

            wait_rd(ev1, zv1, r1)

            @pl.when(jj > 0)
            def _():
                wait_w(wv1, w1)

            comp(ev1, zv1, wv1)
            wr(j0 + 1, wv1, w1)
            return carry

        lax.fori_loop(0, NPAIR, body, 0)
        wait_w(wv0, w0)
        wait_w(wv1, w1)
        pltpu.sync_copy(e_hbm.at[pl.ds(base + NBL * BL, TAIL)],
                        ev0.at[pl.ds(0, TAIL)])
        pltpu.async_copy(zc_hbm.at[idx_v.at[NBL, pl.ds(0, TAIL)]],
                         zv0.at[pl.ds(0, TAIL)], r0).wait()
        wv0[pl.ds(0, 16)] = ev0[pl.ds(0, 16)] / (zv0[pl.ds(0, 16)] + 1e-16)
        pltpu.sync_copy(wv0.at[pl.ds(0, TAIL)],
                        w_hbm.at[pl.ds(base + NBL * BL, TAIL)])

    return k(e, zc, tgt_r)


def kernel(x, edge_index, edge_attr, W1, b1, W2, b2, att_vec):
    src = edge_index[0].astype(jnp.int32)
    tgt = edge_index[1].astype(jnp.int32)
    src_r = jnp.pad(src.reshape(NW, EW), ((0, 0), (0, IDXR * BL - EW))
                    ).reshape(NW, IDXR, BL)
    tgt_r = jnp.pad(tgt.reshape(NW, EW), ((0, 0), (0, IDXR * BL - EW))
                    ).reshape(NW, IDXR, BL)

    xs = _sc_gather(x, src_r)
    wmsg, e = _tc_edge(xs, edge_attr, W1, b1, W2, b2, att_vec)
    a2, z2 = _sc_scatter(wmsg, e, tgt_r)
    outp, zc = _tc_finalize(a2, z2)
    w = _sc_weights(e, zc, tgt_r)
    return outp[:N_NODES], w


# speedup vs baseline: 1.1932x; 1.1884x over previous
import functools

import jax
import jax.numpy as jnp
from jax import lax
from jax.experimental import pallas as pl
from jax.experimental.pallas import tpu as pltpu
from jax.experimental.pallas import tpu_sc as plsc

N_NODES = 10000
N_EDGES = 320000
D_FEAT = 128
D_EDGE = 16

NC = 2
NS = 16
NW = NC * NS
EW = N_EDGES // NW
BL = 128
NBL = EW // BL
TAIL = EW - NBL * BL
NPAIR = NBL // 2
NP = 10240
SROW = NP // NS
IDXR = (EW + BL - 1) // BL + 1


def _mesh():
    return plsc.VectorSubcoreMesh(core_axis_name="c", subcore_axis_name="s",
                                  num_cores=NC, num_subcores=NS)


def _wid():
    return lax.axis_index("s") * NC + lax.axis_index("c")


def _sc_gather(x, src_r):
    @functools.partial(
        pl.kernel,
        out_type=jax.ShapeDtypeStruct((N_EDGES, D_FEAT), jnp.float32),
        mesh=_mesh(),
        scratch_types=[
            pltpu.VMEM((IDXR, BL), jnp.int32),
            pltpu.VMEM((BL, D_FEAT), jnp.float32),
            pltpu.VMEM((BL, D_FEAT), jnp.float32),
            pltpu.SemaphoreType.DMA,
            pltpu.SemaphoreType.DMA,
            pltpu.SemaphoreType.DMA,
            pltpu.SemaphoreType.DMA,
        ],
    )
    def k(x_hbm, src_hbm, xs_hbm, idx_v, rows0, rows1, g0, g1, w0, w1):
        wid = _wid()
        base = wid * EW
        pltpu.sync_copy(src_hbm.at[wid], idx_v)

        def gat(j, buf, sem):
            pltpu.async_copy(x_hbm.at[idx_v.at[j]], buf, sem)

        def wait_g(buf, sem):
            pltpu.make_async_copy(xs_hbm.at[pl.ds(0, BL)], buf, sem).wait()

        def wr(j, buf, sem):
            pltpu.async_copy(buf, xs_hbm.at[pl.ds(base + j * BL, BL)], sem)

        def wait_w(buf, sem):
            pltpu.make_async_copy(buf, xs_hbm.at[pl.ds(0, BL)], sem).wait()

        gat(0, rows0, g0)

        def body(jj, carry):
            j0 = 2 * jj
            gat(j0 + 1, rows1, g1)
            wait_g(rows0, g0)
            wr(j0, rows0, w0)

            @pl.when(jj < NPAIR - 1)
            def _():
                wait_w(rows0, w0)
                gat(j0 + 2, rows0, g0)

            wait_g(rows1, g1)
            wr(j0 + 1, rows1, w1)
            wait_w(rows1, w1)
            return carry

        lax.fori_loop(0, NPAIR, body, 0)
        wait_w(rows0, w0)
        tg = pltpu.async_copy(
            x_hbm.at[idx_v.at[NBL, pl.ds(0, TAIL)]],
            rows0.at[pl.ds(0, TAIL)], g0)
        tg.wait()
        pltpu.sync_copy(rows0.at[pl.ds(0, TAIL)],
                        xs_hbm.at[pl.ds(base + NBL * BL, TAIL)])

    return k(x, src_r)


def _tc_edge(xs, edge_attr, W1, b1, W2, b2, att_vec):
    BE = 1280
    grid = (N_EDGES // BE,)

    def body(xs_ref, ea_ref, w1_ref, b1_ref, w2_ref, b2_ref, att_ref,
             wmsg_ref, e_ref):
        ones_bc = jnp.ones((D_EDGE, D_FEAT), jnp.float32)
        ea = ea_ref[...]
        n2bc = jnp.dot(ea * ea, ones_bc, preferred_element_type=jnp.float32)
        rinv = 1.0 / (jnp.sqrt(n2bc) + 1e-8)
        g = jnp.dot(ea, w1_ref[...], preferred_element_type=jnp.float32)
        h = jnp.tanh(g * rinv + b1_ref[...])
        filt = (jnp.dot(h.astype(jnp.bfloat16),
                        w2_ref[...].astype(jnp.bfloat16),
                        preferred_element_type=jnp.float32)
                + b2_ref[...])
        msg = xs_ref[...] * filt
        s = jnp.dot(msg, att_ref[...], preferred_element_type=jnp.float32)
        e = jnp.exp(s)
        ebc = jnp.dot(e, jnp.ones((1, D_FEAT), jnp.float32),
                      preferred_element_type=jnp.float32)
        wmsg_ref[...] = msg * ebc
        e_ref[...] = e[:, 0][None, None, :]

    wmsg, e = pl.pallas_call(
        body,
        grid=grid,
        in_specs=[
            pl.BlockSpec((BE, D_FEAT), lambda i: (i, 0)),
            pl.BlockSpec((BE, D_EDGE), lambda i: (i, 0)),
            pl.BlockSpec((D_EDGE, D_FEAT), lambda i: (0, 0)),
            pl.BlockSpec((D_FEAT,), lambda i: (0,)),
            pl.BlockSpec((D_FEAT, D_FEAT), lambda i: (0, 0)),
            pl.BlockSpec((D_FEAT,), lambda i: (0,)),
            pl.BlockSpec((D_FEAT, 1), lambda i: (0, 0)),
        ],
        out_specs=[
            pl.BlockSpec((BE, D_FEAT), lambda i: (i, 0)),
            pl.BlockSpec((1, 1, BE), lambda i: (i, 0, 0)),
        ],
        out_shape=[
            jax.ShapeDtypeStruct((N_EDGES, D_FEAT), jnp.float32),
            jax.ShapeDtypeStruct((N_EDGES // BE, 1, BE), jnp.float32),
        ],
    )(xs, edge_attr, W1, b1, W2, b2, att_vec)
    return wmsg, e.reshape(N_EDGES)


def _sc_scatter(wmsg, e, tgt_r):
    @functools.partial(
        pl.kernel,
        out_type=[
            jax.ShapeDtypeStruct((NC, NP, D_FEAT), jnp.float32),
            jax.ShapeDtypeStruct((NC, NP), jnp.float32),
        ],
        mesh=_mesh(),
        scratch_types=[
            pltpu.VMEM((IDXR, BL), jnp.int32),
            pltpu.VMEM((BL, D_FEAT), jnp.float32),
            pltpu.VMEM((BL, D_FEAT), jnp.float32),
            pltpu.VMEM((BL,), jnp.float32),
            pltpu.VMEM((BL,), jnp.float32),
            pltpu.VMEM((32, D_FEAT), jnp.float32),
            pltpu.VMEM((SROW,), jnp.float32),
            pltpu.VMEM_SHARED((NP, D_FEAT), jnp.float32),
            pltpu.VMEM_SHARED((NP,), jnp.float32),
            pltpu.SemaphoreType.DMA,
            pltpu.SemaphoreType.DMA,
        ],
    )
    def k(wmsg_hbm, e_hbm, tgt_hbm, a2_hbm, z2_hbm,
          idx_v, m0, m1, e0, e1, zb_v, zb1_v, a_sh, z_sh, r0, r1):
        c = lax.axis_index("c")
        s = lax.axis_index("s")
        wid = s * NC + c
        base = wid * EW

        zeros16 = jnp.zeros((16,), jnp.float32)

        def z0(i, carry):
            def z1(kk, carry2):
                zb_v[i, pl.ds(kk * 16, 16)] = zeros16
                return carry2
            return lax.fori_loop(0, 8, z1, carry)

        lax.fori_loop(0, 32, z0, 0)

        def z2(i, carry):
            zb1_v[pl.ds(i * 16, 16)] = zeros16
            return carry

        lax.fori_loop(0, SROW // 16, z2, 0)

        def za(t, carry):
            pltpu.sync_copy(zb_v, a_sh.at[pl.ds(s * SROW + t * 32, 32)])
            return carry

        lax.fori_loop(0, SROW // 32, za, 0)
        pltpu.sync_copy(zb1_v, z_sh.at[pl.ds(s * SROW, SROW)])

        pltpu.sync_copy(tgt_hbm.at[wid], idx_v)
        plsc.subcore_barrier()

        def rd(j, mb, eb, sem):
            pltpu.async_copy(wmsg_hbm.at[pl.ds(base + j * BL, BL)], mb, sem)
            pltpu.async_copy(e_hbm.at[pl.ds(base + j * BL, BL)], eb, sem)

        def wait_rd(mb, eb, sem):
            pltpu.make_async_copy(wmsg_hbm.at[pl.ds(0, BL)], mb, sem).wait()
            pltpu.make_async_copy(e_hbm.at[pl.ds(0, BL)], eb, sem).wait()

        def scat(j, mb, eb):
            pltpu.sync_copy(mb, a_sh.at[idx_v.at[j]], add=True)
            pltpu.sync_copy(eb, z_sh.at[idx_v.at[j]], add=True)

        rd(0, m0, e0, r0)

        def body(jj, carry):
            j0 = 2 * jj
            rd(j0 + 1, m1, e1, r1)
            wait_rd(m0, e0, r0)
            scat(j0, m0, e0)

            @pl.when(jj < NPAIR - 1)
            def _():
                rd(j0 + 2, m0, e0, r0)

            wait_rd(m1, e1, r1)
            scat(j0 + 1, m1, e1)
            return carry

        lax.fori_loop(0, NPAIR, body, 0)
        pltpu.sync_copy(wmsg_hbm.at[pl.ds(base + NBL * BL, TAIL)],
                        m0.at[pl.ds(0, TAIL)])
        pltpu.sync_copy(e_hbm.at[pl.ds(base + NBL * BL, TAIL)],
                        e0.at[pl.ds(0, TAIL)])
        pltpu.sync_copy(m0.at[pl.ds(0, TAIL)],
                        a_sh.at[idx_v.at[NBL, pl.ds(0, TAIL)]], add=True)
        pltpu.sync_copy(e0.at[pl.ds(0, TAIL)],
                        z_sh.at[idx_v.at[NBL, pl.ds(0, TAIL)]], add=True)

        plsc.subcore_barrier()
        pltpu.sync_copy(a_sh.at[pl.ds(s * SROW, SROW)],
                        a2_hbm.at[c, pl.ds(s * SROW, SROW)])
        pltpu.sync_copy(z_sh.at[pl.ds(s * SROW, SROW)],
                        z2_hbm.at[c, pl.ds(s * SROW, SROW)])

    return k(wmsg, e, tgt_r)


def _tc_finalize(a2, z2):
    BN = 1024
    grid = (NP // BN,)

    def body(a_ref, z_ref, out_ref, zc_ref):
        a = a_ref[0] + a_ref[1]
        z = z_ref[0] + z_ref[1]
        zc_ref[...] = z
        out_ref[...] = a / (z[:, None] + 1e-16)

    outp, zc = pl.pallas_call(
        body,
        grid=grid,
        in_specs=[
            pl.BlockSpec((NC, BN, D_FEAT), lambda i: (0, i, 0)),
            pl.BlockSpec((NC, BN), lambda i: (0, i)),
        ],
        out_specs=[
            pl.BlockSpec((BN, D_FEAT), lambda i: (i, 0)),
            pl.BlockSpec((BN,), lambda i: (i,)),
        ],
        out_shape=[
            jax.ShapeDtypeStruct((NP, D_FEAT), jnp.float32),
            jax.ShapeDtypeStruct((NP,), jnp.float32),
        ],
    )(a2, z2)
    return outp, zc


def _sc_weights(e, zc, tgt_r):
    @functools.partial(
        pl.kernel,
        out_type=jax.ShapeDtypeStruct((N_EDGES,), jnp.float32),
        mesh=_mesh(),
        scratch_types=[
            pltpu.VMEM((IDXR, BL), jnp.int32),
            pltpu.VMEM((BL,), jnp.float32),
            pltpu.VMEM((BL,), jnp.float32),
            pltpu.VMEM((BL,), jnp.float32),
            pltpu.VMEM((BL,), jnp.float32),
            pltpu.VMEM((BL,), jnp.float32),
            pltpu.VMEM((BL,), jnp.float32),
            pltpu.SemaphoreType.DMA,
            pltpu.SemaphoreType.DMA,
            pltpu.SemaphoreType.DMA,
            pltpu.SemaphoreType.DMA,
        ],
    )
    def k(e_hbm, zc_hbm, tgt_hbm, w_hbm,
          idx_v, ev0, ev1, zv0, zv1, wv0, wv1, r0, r1, w0, w1):
        wid = _wid()
        base = wid * EW
        pltpu.sync_copy(tgt_hbm.at[wid], idx_v)

        def rd(j, eb, zb, sem):
            pltpu.async_copy(e_hbm.at[pl.ds(base + j * BL, BL)], eb, sem)
            pltpu.async_copy(zc_hbm.at[idx_v.at[j]], zb, sem)

        def wait_rd(eb, zb, sem):
            pltpu.make_async_copy(e_hbm.at[pl.ds(0, BL)], eb, sem).wait()
            pltpu.make_async_copy(e_hbm.at[pl.ds(0, BL)], zb, sem).wait()

        def comp(eb, zb, wb):
            for kk in range(BL // 16):
                sl = pl.ds(kk * 16, 16)
                wb[sl] = eb[sl] / (zb[sl] + 1e-16)

        def wr(j, wb, sem):
            pltpu.async_copy(wb, w_hbm.at[pl.ds(base + j * BL, BL)], sem)

        def wait_w(wb, sem):
            pltpu.make_async_copy(wb, w_hbm.at[pl.ds(0, BL)], sem).wait()

        rd(0, ev0, zv0, r0)

        def body(jj, carry):
            j0 = 2 * jj
            rd(j0 + 1, ev1, zv1, r1)
            wait_rd(ev0, zv0, r0)

            @pl.when(jj > 0)
            def _():
                wait_w(wv0, w0)

            comp(ev0, zv0, wv0)
            wr(j0, wv0, w0)

            @pl.when(jj < NPAIR - 1)
            def _():
                rd(j0 + 2, ev0, zv0, r0)

            wait_rd(ev1, zv1, r1)

            @pl.when(jj > 0)
            def _():
                wait_w(wv1, w1)

            comp(ev1, zv1, wv1)
            wr(j0 + 1, wv1, w1)
            return carry

        lax.fori_loop(0, NPAIR, body, 0)
        wait_w(wv0, w0)
        wait_w(wv1, w1)
        pltpu.sync_copy(e_hbm.at[pl.ds(base + NBL * BL, TAIL)],
                        ev0.at[pl.ds(0, TAIL)])
        pltpu.async_copy(zc_hbm.at[idx_v.at[NBL, pl.ds(0, TAIL)]],
                         zv0.at[pl.ds(0, TAIL)], r0).wait()
        wv0[pl.ds(0, 16)] = ev0[pl.ds(0, 16)] / (zv0[pl.ds(0, 16)] + 1e-16)
        pltpu.sync_copy(wv0.at[pl.ds(0, TAIL)],
                        w_hbm.at[pl.ds(base + NBL * BL, TAIL)])

    return k(e, zc, tgt_r)


def kernel(x, edge_index, edge_attr, W1, b1, W2, b2, att_vec):
    src = edge_index[0].astype(jnp.int32)
    tgt = edge_index[1].astype(jnp.int32)
    src_r = jnp.pad(src.reshape(NW, EW), ((0, 0), (0, IDXR * BL - EW))
                    ).reshape(NW, IDXR, BL)
    tgt_r = jnp.pad(tgt.reshape(NW, EW), ((0, 0), (0, IDXR * BL - EW))
                    ).reshape(NW, IDXR, BL)

    xs = _sc_gather(x, src_r)
    wmsg, e = _tc_edge(xs, edge_attr, W1, b1, W2, b2, att_vec)
    a2, z2 = _sc_scatter(wmsg, e, tgt_r)
    outp, zc = _tc_finalize(a2, z2)
    w = _sc_weights(e, zc, tgt_r)
    return outp[:N_NODES], w
